# Initial kernel scaffold; baseline (speedup 1.0000x reference)
import jax
import jax.numpy as jnp
from jax.experimental import pallas as pl


def kernel(src, length, aa_emb, pos_emb, gamma, beta, W, b):
    raise NotImplementedError("write your pallas kernel here")


# speedup vs baseline: 6.1677x; 6.1677x over previous
import functools

import jax
import jax.numpy as jnp
from jax import lax
from jax.experimental import pallas as pl
from jax.experimental.pallas import tpu as pltpu
from jax.experimental.pallas import tpu_sc as plsc

VOCAB = 22
EMB = 16
MAXLEN = 35
PAD = 21
EPS = 1e-12
HID2 = 32


def _table_body(va_ref, vp_ref, g_ref, be_ref, w_ref, b_ref, out_ref):
    e = va_ref[...] + vp_ref[...]
    mu = jnp.mean(e, axis=1, keepdims=True)
    var = jnp.mean((e - mu) ** 2, axis=1, keepdims=True)
    normed = (e - mu) / jnp.sqrt(var + EPS) * g_ref[...] + be_ref[...]
    out_ref[...] = (
        jnp.dot(normed, w_ref[...], preferred_element_type=jnp.float32)
        + b_ref[...]
    )


def _idx_body(src_ref, out_ref):
    s = src_ref[...]
    mask = s != PAD
    mf = mask.astype(jnp.float32)
    ll = s.shape[1]
    k = lax.broadcasted_iota(jnp.int32, (ll, ll), 0)
    j = lax.broadcasted_iota(jnp.int32, (ll, ll), 1)
    tri = (k <= j).astype(jnp.float32)
    csum = jnp.dot(mf, tri, preferred_element_type=jnp.float32)
    pid = csum.astype(jnp.int32) * mask.astype(jnp.int32)
    out_ref[...] = s * MAXLEN + pid


@functools.partial(jax.jit, static_argnums=(0, 1))
def _sc_gather(n_rows, d, table, idx2d):
    NC, NS = 2, 16
    nw = NC * NS
    per_w = n_rows // nw
    ch = 128
    n_ch = per_w // ch
    mesh = plsc.VectorSubcoreMesh(core_axis_name="c", subcore_axis_name="s")

    @functools.partial(
        pl.kernel,
        mesh=mesh,
        out_type=jax.ShapeDtypeStruct((n_rows, d), jnp.float32),
        scratch_types=[
            pltpu.VMEM((n_ch, ch), jnp.int32),
            pltpu.VMEM((ch, d), jnp.float32),
            pltpu.SemaphoreType.DMA,
        ],
        compiler_params=pltpu.CompilerParams(use_tc_tiling_on_sc=False),
    )
    def k(table_hbm, idx_hbm, out_hbm, idx_v, buf_v, sem):
        wid = lax.axis_index("s") * NC + lax.axis_index("c")
        base = wid * per_w
        pltpu.sync_copy(idx_hbm.at[pl.ds(wid * n_ch, n_ch)], idx_v)

        def body(jc, carry):
            pltpu.async_copy(table_hbm.at[idx_v.at[jc]], buf_v, sem).wait()
            pltpu.sync_copy(buf_v, out_hbm.at[pl.ds(base + jc * ch, ch)])
            return carry

        lax.fori_loop(0, n_ch, body, 0)

    return k(table, idx2d)


def kernel(src, length, aa_emb, pos_emb, gamma, beta, W, b):
    bb, ll = src.shape
    va = jnp.repeat(aa_emb, MAXLEN, axis=0)
    vp = jnp.tile(pos_emb, (VOCAB, 1))
    table = pl.pallas_call(
        _table_body,
        out_shape=jax.ShapeDtypeStruct((VOCAB * MAXLEN, HID2), jnp.float32),
    )(va, vp, gamma.reshape(1, EMB), beta.reshape(1, EMB), W, b)

    grid = 16
    idx = pl.pallas_call(
        _idx_body,
        grid=(grid,),
        in_specs=[pl.BlockSpec((bb // grid, ll), lambda i: (i, 0))],
        out_specs=pl.BlockSpec((bb // grid, ll), lambda i: (i, 0)),
        out_shape=jax.ShapeDtypeStruct((bb, ll), jnp.int32),
    )(src)

    idx2d = idx.reshape(-1, 128)
    out = _sc_gather(bb * ll, HID2, table, idx2d)
    return out.reshape(bb, ll, HID2)
